# baseline (speedup 1.0000x reference)
import jax
import jax.numpy as jnp
from jax import lax
from jax.experimental import pallas as pl
from jax.experimental.pallas import tpu as pltpu
from jax.experimental.pallas import tpu_sc as plsc

_T = 64
_K = 8
_E = 64
_LANES = 16
_ROW = 128

_CHUNK = 128
_NWORK = (_T * _K) // _CHUNK


def _routing_scatter_body(sel_hbm, rw_hbm, zeros_hbm, w_hbm,
                          sel_v, rw_v, idx_v, w_sh):
    c = lax.axis_index("c")
    s = lax.axis_index("s")

    @pl.when((c == 0) & (s == 0))
    def _zero():
        pltpu.sync_copy(zeros_hbm, w_sh)

    plsc.subcore_barrier()

    @pl.when((c == 0) & (s < _NWORK))
    def _scatter():
        base = s * _CHUNK
        pltpu.sync_copy(sel_hbm.at[pl.ds(base, _CHUNK)], sel_v)
        pltpu.sync_copy(rw_hbm.at[pl.ds(base, _CHUNK)], rw_v)

        lane = lax.broadcasted_iota(jnp.int32, (_LANES,), 0)
        lane_tok = jnp.where(lane >= _K, 1, 0)

        def idx_body(j, carry):
            off = j * _LANES
            sel = sel_v[pl.ds(off, _LANES)]
            t = (base + off) // _K + lane_tok
            idx_v[pl.ds(off, _LANES)] = sel * _ROW + t
            return carry

        lax.fori_loop(0, _CHUNK // _LANES, idx_body, 0)
        pltpu.sync_copy(rw_v, w_sh.at[idx_v], add=True)

    plsc.subcore_barrier()

    @pl.when((c == 0) & (s == 0))
    def _writeout():
        pltpu.sync_copy(w_sh, w_hbm)


def _routing_weights_sc(selected_experts, routing_weights):
    sel_flat = selected_experts.reshape(-1)
    rw_flat = routing_weights.reshape(-1)
    zeros = jnp.zeros((_E * _ROW,), jnp.float32)
    mesh = plsc.VectorSubcoreMesh(core_axis_name="c", subcore_axis_name="s")
    w = pl.kernel(
        _routing_scatter_body,
        mesh=mesh,
        out_type=jax.ShapeDtypeStruct((_E * _ROW,), jnp.float32),
        scratch_types=[
            pltpu.VMEM((_CHUNK,), jnp.int32),
            pltpu.VMEM((_CHUNK,), jnp.float32),
            pltpu.VMEM((_CHUNK,), jnp.int32),
            pltpu.VMEM_SHARED((_E * _ROW,), jnp.float32),
        ],
    )(sel_flat, rw_flat, zeros)
    return w.reshape(_E, _ROW)


def _moe_body(hidden_ref, w_ref, gate_ref, up_ref, down_ref, out_ref,
              xt_ref, acc_ref):
    e = pl.program_id(0)

    @pl.when(e == 0)
    def _prep():
        xt_ref[...] = hidden_ref[...].T

    xt = xt_ref[...]
    gt = jax.lax.dot_general(gate_ref[...], xt, (((1,), (0,)), ((), ())),
                             preferred_element_type=jnp.float32)
    ut = jax.lax.dot_general(up_ref[...], xt, (((1,), (0,)), ((), ())),
                             preferred_element_type=jnp.float32)
    ht = gt * jax.nn.sigmoid(gt) * ut
    dt = jax.lax.dot_general(down_ref[...], ht, (((1,), (0,)), ((), ())),
                             preferred_element_type=jnp.float32)
    wrow = w_ref[pl.ds(e, 1), 0:_T]
    contrib = wrow * dt

    @pl.when(e == 0)
    def _init():
        acc_ref[...] = contrib

    @pl.when(e != 0)
    def _acc():
        acc_ref[...] += contrib

    @pl.when(e == pl.num_programs(0) - 1)
    def _fin():
        out_ref[...] = acc_ref[...].T


def kernel(hidden_states, routing_weights, selected_experts, num_experts,
           gate_proj, up_proj, down_proj):
    T, H = hidden_states.shape
    E, I, _ = gate_proj.shape
    w = _routing_weights_sc(selected_experts, routing_weights)
    return pl.pallas_call(
        _moe_body,
        grid=(E,),
        in_specs=[
            pl.BlockSpec((T, H), lambda e: (0, 0)),
            pl.BlockSpec((E, _ROW), lambda e: (0, 0)),
            pl.BlockSpec((None, I, H), lambda e: (e, 0, 0)),
            pl.BlockSpec((None, I, H), lambda e: (e, 0, 0)),
            pl.BlockSpec((None, H, I), lambda e: (e, 0, 0)),
        ],
        out_specs=pl.BlockSpec((T, H), lambda e: (0, 0)),
        out_shape=jax.ShapeDtypeStruct((T, H), jnp.float32),
        scratch_shapes=[
            pltpu.VMEM((H, T), jnp.float32),
            pltpu.VMEM((H, T), jnp.float32),
        ],
    )(hidden_states, w, gate_proj, up_proj, down_proj)


# speedup vs baseline: 1.0323x; 1.0323x over previous
import jax
import jax.numpy as jnp
from jax import lax
from jax.experimental import pallas as pl
from jax.experimental.pallas import tpu as pltpu
from jax.experimental.pallas import tpu_sc as plsc

_T = 64
_K = 8
_E = 64
_LANES = 16
_ROW = 128

_CHUNK = 128
_NWORK = (_T * _K) // _CHUNK


def _routing_scatter_body(sel_hbm, rw_hbm, zeros_hbm, w_hbm,
                          sel_v, rw_v, idx_v, w_sh):
    c = lax.axis_index("c")
    s = lax.axis_index("s")

    @pl.when((c == 0) & (s == 0))
    def _zero():
        pltpu.sync_copy(zeros_hbm, w_sh)

    plsc.subcore_barrier()

    @pl.when((c == 0) & (s < _NWORK))
    def _scatter():
        base = s * _CHUNK
        pltpu.sync_copy(sel_hbm.at[pl.ds(base, _CHUNK)], sel_v)
        pltpu.sync_copy(rw_hbm.at[pl.ds(base, _CHUNK)], rw_v)

        lane = lax.broadcasted_iota(jnp.int32, (_LANES,), 0)
        lane_tok = jnp.where(lane >= _K, 1, 0)

        def idx_body(j, carry):
            off = j * _LANES
            sel = sel_v[pl.ds(off, _LANES)]
            t = (base + off) // _K + lane_tok
            idx_v[pl.ds(off, _LANES)] = sel * _ROW + t
            return carry

        lax.fori_loop(0, _CHUNK // _LANES, idx_body, 0)
        pltpu.sync_copy(rw_v, w_sh.at[idx_v], add=True)

    plsc.subcore_barrier()

    @pl.when((c == 0) & (s == 0))
    def _writeout():
        pltpu.sync_copy(w_sh, w_hbm)


def _routing_weights_sc(selected_experts, routing_weights):
    sel_flat = selected_experts.reshape(-1)
    rw_flat = routing_weights.reshape(-1)
    zeros = jnp.zeros((_E * _ROW,), jnp.float32)
    mesh = plsc.VectorSubcoreMesh(core_axis_name="c", subcore_axis_name="s")
    w = pl.kernel(
        _routing_scatter_body,
        mesh=mesh,
        out_type=jax.ShapeDtypeStruct((_E * _ROW,), jnp.float32),
        scratch_types=[
            pltpu.VMEM((_CHUNK,), jnp.int32),
            pltpu.VMEM((_CHUNK,), jnp.float32),
            pltpu.VMEM((_CHUNK,), jnp.int32),
            pltpu.VMEM_SHARED((_E * _ROW,), jnp.float32),
        ],
    )(sel_flat, rw_flat, zeros)
    return w.reshape(_E, _ROW)


def _moe_body(hidden_ref, w_ref, gate_ref, up_ref, down_ref, out_ref):
    e = pl.program_id(0)
    x = hidden_ref[...]
    g = jax.lax.dot_general(x, gate_ref[...], (((1,), (1,)), ((), ())),
                            preferred_element_type=jnp.float32)
    u = jax.lax.dot_general(x, up_ref[...], (((1,), (1,)), ((), ())),
                            preferred_element_type=jnp.float32)
    h = g * jax.nn.sigmoid(g) * u
    d = jax.lax.dot_general(h, down_ref[...], (((1,), (1,)), ((), ())),
                            preferred_element_type=jnp.float32)
    wrow = w_ref[pl.ds(e, 1), 0:_T]
    ident = (lax.broadcasted_iota(jnp.int32, (_T, _T), 0)
             == lax.broadcasted_iota(jnp.int32, (_T, _T), 1)
             ).astype(jnp.float32)
    wcol = jax.lax.dot_general(ident, wrow, (((1,), (1,)), ((), ())),
                               preferred_element_type=jnp.float32)
    contrib = wcol * d

    @pl.when(e == 0)
    def _init():
        out_ref[...] = contrib

    @pl.when(e != 0)
    def _acc():
        out_ref[...] += contrib


def kernel(hidden_states, routing_weights, selected_experts, num_experts,
           gate_proj, up_proj, down_proj):
    T, H = hidden_states.shape
    E, I, _ = gate_proj.shape
    w = _routing_weights_sc(selected_experts, routing_weights)
    return pl.pallas_call(
        _moe_body,
        grid=(E,),
        in_specs=[
            pl.BlockSpec((T, H), lambda e: (0, 0)),
            pl.BlockSpec((E, _ROW), lambda e: (0, 0)),
            pl.BlockSpec((None, I, H), lambda e: (e, 0, 0)),
            pl.BlockSpec((None, I, H), lambda e: (e, 0, 0)),
            pl.BlockSpec((None, H, I), lambda e: (e, 0, 0)),
        ],
        out_specs=pl.BlockSpec((T, H), lambda e: (0, 0)),
        out_shape=jax.ShapeDtypeStruct((T, H), jnp.float32),
    )(hidden_states, w, gate_proj, up_proj, down_proj)


# speedup vs baseline: 1.0382x; 1.0058x over previous
import jax
import jax.numpy as jnp
from jax import lax
from jax.experimental import pallas as pl
from jax.experimental.pallas import tpu as pltpu
from jax.experimental.pallas import tpu_sc as plsc

_T = 64
_K = 8
_E = 64
_LANES = 16
_ROW = 128

_CHUNK = 128
_NWORK = (_T * _K) // _CHUNK


def _routing_scatter_body(sel_hbm, rw_hbm, zeros_hbm, w_hbm,
                          sel_v, rw_v, idx_v, w_sh):
    c = lax.axis_index("c")
    s = lax.axis_index("s")

    @pl.when((c == 0) & (s == 0))
    def _zero():
        pltpu.sync_copy(zeros_hbm, w_sh)

    plsc.subcore_barrier()

    @pl.when((c == 0) & (s < _NWORK))
    def _scatter():
        base = s * _CHUNK
        pltpu.sync_copy(sel_hbm.at[pl.ds(base, _CHUNK)], sel_v)
        pltpu.sync_copy(rw_hbm.at[pl.ds(base, _CHUNK)], rw_v)

        lane = lax.broadcasted_iota(jnp.int32, (_LANES,), 0)
        lane_tok = jnp.where(lane >= _K, 1, 0)

        def idx_body(j, carry):
            off = j * _LANES
            sel = sel_v[pl.ds(off, _LANES)]
            t = (base + off) // _K + lane_tok
            idx_v[pl.ds(off, _LANES)] = sel * _ROW + t
            return carry

        lax.fori_loop(0, _CHUNK // _LANES, idx_body, 0)
        pltpu.sync_copy(rw_v, w_sh.at[idx_v], add=True)

    plsc.subcore_barrier()

    @pl.when((c == 0) & (s == 0))
    def _writeout():
        pltpu.sync_copy(w_sh, w_hbm)


def _routing_weights_sc(selected_experts, routing_weights):
    sel_flat = selected_experts.reshape(-1)
    rw_flat = routing_weights.reshape(-1)
    zeros = jnp.zeros((_E * _ROW,), jnp.float32)
    mesh = plsc.VectorSubcoreMesh(core_axis_name="c", subcore_axis_name="s",
                                  num_cores=1)
    w = pl.kernel(
        _routing_scatter_body,
        mesh=mesh,
        out_type=jax.ShapeDtypeStruct((_E * _ROW,), jnp.float32),
        scratch_types=[
            pltpu.VMEM((_CHUNK,), jnp.int32),
            pltpu.VMEM((_CHUNK,), jnp.float32),
            pltpu.VMEM((_CHUNK,), jnp.int32),
            pltpu.VMEM_SHARED((_E * _ROW,), jnp.float32),
        ],
    )(sel_flat, rw_flat, zeros)
    return w.reshape(_E, _ROW)


def _moe_body(hidden_ref, w_ref, gate_ref, up_ref, down_ref, out_ref):
    e = pl.program_id(0)
    x = hidden_ref[...]
    g = jax.lax.dot_general(x, gate_ref[...], (((1,), (1,)), ((), ())),
                            preferred_element_type=jnp.float32)
    u = jax.lax.dot_general(x, up_ref[...], (((1,), (1,)), ((), ())),
                            preferred_element_type=jnp.float32)
    h = g * jax.nn.sigmoid(g) * u
    d = jax.lax.dot_general(h, down_ref[...], (((1,), (1,)), ((), ())),
                            preferred_element_type=jnp.float32)
    wrow = w_ref[pl.ds(e, 1), 0:_T]
    ident = (lax.broadcasted_iota(jnp.int32, (_T, _T), 0)
             == lax.broadcasted_iota(jnp.int32, (_T, _T), 1)
             ).astype(jnp.float32)
    wcol = jax.lax.dot_general(ident, wrow, (((1,), (1,)), ((), ())),
                               preferred_element_type=jnp.float32)
    contrib = wcol * d

    @pl.when(e == 0)
    def _init():
        out_ref[...] = contrib

    @pl.when(e != 0)
    def _acc():
        out_ref[...] += contrib


def kernel(hidden_states, routing_weights, selected_experts, num_experts,
           gate_proj, up_proj, down_proj):
    T, H = hidden_states.shape
    E, I, _ = gate_proj.shape
    w = _routing_weights_sc(selected_experts, routing_weights)
    return pl.pallas_call(
        _moe_body,
        grid=(E,),
        in_specs=[
            pl.BlockSpec((T, H), lambda e: (0, 0)),
            pl.BlockSpec((E, _ROW), lambda e: (0, 0)),
            pl.BlockSpec((None, I, H), lambda e: (e, 0, 0)),
            pl.BlockSpec((None, I, H), lambda e: (e, 0, 0)),
            pl.BlockSpec((None, H, I), lambda e: (e, 0, 0)),
        ],
        out_specs=pl.BlockSpec((T, H), lambda e: (0, 0)),
        out_shape=jax.ShapeDtypeStruct((T, H), jnp.float32),
    )(hidden_states, w, gate_proj, up_proj, down_proj)


# speedup vs baseline: 1.0427x; 1.0042x over previous
import jax
import jax.numpy as jnp
from jax import lax
from jax.experimental import pallas as pl
from jax.experimental.pallas import tpu as pltpu
from jax.experimental.pallas import tpu_sc as plsc

_T = 64
_K = 8
_E = 64
_LANES = 16
_ROW = 128

_CHUNK = 128
_NWORK = (_T * _K) // _CHUNK


def _routing_scatter_body(sel_hbm, rw_hbm, zeros_hbm, w_hbm,
                          sel_v, rw_v, idx_v, w_sh):
    c = lax.axis_index("c")
    s = lax.axis_index("s")

    @pl.when((c == 0) & (s == 0))
    def _zero():
        pltpu.sync_copy(zeros_hbm, w_sh)

    plsc.subcore_barrier()

    @pl.when((c == 0) & (s < _NWORK))
    def _scatter():
        base = s * _CHUNK
        pltpu.sync_copy(sel_hbm.at[pl.ds(base, _CHUNK)], sel_v)
        pltpu.sync_copy(rw_hbm.at[pl.ds(base, _CHUNK)], rw_v)

        lane = lax.broadcasted_iota(jnp.int32, (_LANES,), 0)
        lane_tok = jnp.where(lane >= _K, 1, 0)

        def idx_body(j, carry):
            off = j * _LANES
            sel = sel_v[pl.ds(off, _LANES)]
            t = (base + off) // _K + lane_tok
            idx_v[pl.ds(off, _LANES)] = sel * _ROW + t
            return carry

        lax.fori_loop(0, _CHUNK // _LANES, idx_body, 0)
        pltpu.sync_copy(rw_v, w_sh.at[idx_v], add=True)

    plsc.subcore_barrier()

    @pl.when((c == 0) & (s == 0))
    def _writeout():
        pltpu.sync_copy(w_sh, w_hbm)


def _routing_weights_sc(selected_experts, routing_weights):
    sel_flat = selected_experts.reshape(-1)
    rw_flat = routing_weights.reshape(-1)
    zeros = jnp.zeros((_E * _ROW,), jnp.float32)
    mesh = plsc.VectorSubcoreMesh(core_axis_name="c", subcore_axis_name="s",
                                  num_cores=1)
    w = pl.kernel(
        _routing_scatter_body,
        mesh=mesh,
        out_type=jax.ShapeDtypeStruct((_E * _ROW,), jnp.float32),
        scratch_types=[
            pltpu.VMEM((_CHUNK,), jnp.int32),
            pltpu.VMEM((_CHUNK,), jnp.float32),
            pltpu.VMEM((_CHUNK,), jnp.int32),
            pltpu.VMEM_SHARED((_E * _ROW,), jnp.float32),
        ],
    )(sel_flat, rw_flat, zeros)
    return w.reshape(_E, _ROW)


def _moe_body(hidden_ref, ident_ref, w_ref, gate_ref, up_ref, down_ref,
              out_ref):
    e = pl.program_id(0)
    x = hidden_ref[...]
    g = jax.lax.dot_general(x, gate_ref[...], (((1,), (1,)), ((), ())),
                            preferred_element_type=jnp.float32)
    u = jax.lax.dot_general(x, up_ref[...], (((1,), (1,)), ((), ())),
                            preferred_element_type=jnp.float32)
    h = g * jax.nn.sigmoid(g) * u
    d = jax.lax.dot_general(h, down_ref[...], (((1,), (1,)), ((), ())),
                            preferred_element_type=jnp.float32)
    wrow = w_ref[pl.ds(e, 1), 0:_T]
    wcol = jax.lax.dot_general(ident_ref[...], wrow,
                               (((1,), (1,)), ((), ())),
                               preferred_element_type=jnp.float32)
    contrib = wcol * d

    @pl.when(e == 0)
    def _init():
        out_ref[...] = contrib

    @pl.when(e != 0)
    def _acc():
        out_ref[...] += contrib


def kernel(hidden_states, routing_weights, selected_experts, num_experts,
           gate_proj, up_proj, down_proj):
    T, H = hidden_states.shape
    E, I, _ = gate_proj.shape
    w = _routing_weights_sc(selected_experts, routing_weights)
    ident = jnp.eye(T, dtype=jnp.float32)
    return pl.pallas_call(
        _moe_body,
        grid=(E,),
        in_specs=[
            pl.BlockSpec((T, H), lambda e: (0, 0)),
            pl.BlockSpec((T, T), lambda e: (0, 0)),
            pl.BlockSpec((E, _ROW), lambda e: (0, 0)),
            pl.BlockSpec((None, I, H), lambda e: (e, 0, 0)),
            pl.BlockSpec((None, I, H), lambda e: (e, 0, 0)),
            pl.BlockSpec((None, H, I), lambda e: (e, 0, 0)),
        ],
        out_specs=pl.BlockSpec((T, H), lambda e: (0, 0)),
        out_shape=jax.ShapeDtypeStruct((T, H), jnp.float32),
    )(hidden_states, ident, w, gate_proj, up_proj, down_proj)


# speedup vs baseline: 1.0473x; 1.0045x over previous
import jax
import jax.numpy as jnp
from jax import lax
from jax.experimental import pallas as pl
from jax.experimental.pallas import tpu as pltpu
from jax.experimental.pallas import tpu_sc as plsc

_T = 64
_K = 8
_E = 64
_LANES = 16
_ROW = 128

_CHUNK = 32
_NWORK = (_T * _K) // _CHUNK


def _routing_scatter_body(sel_hbm, rw_hbm, zeros_hbm, w_hbm,
                          sel_v, rw_v, idx_v, w_sh):
    c = lax.axis_index("c")
    s = lax.axis_index("s")

    @pl.when((c == 0) & (s == 0))
    def _zero():
        pltpu.sync_copy(zeros_hbm, w_sh)

    plsc.subcore_barrier()

    @pl.when((c == 0) & (s < _NWORK))
    def _scatter():
        base = s * _CHUNK
        pltpu.sync_copy(sel_hbm.at[pl.ds(base, _CHUNK)], sel_v)
        pltpu.sync_copy(rw_hbm.at[pl.ds(base, _CHUNK)], rw_v)

        lane = lax.broadcasted_iota(jnp.int32, (_LANES,), 0)
        lane_tok = jnp.where(lane >= _K, 1, 0)

        def idx_body(j, carry):
            off = j * _LANES
            sel = sel_v[pl.ds(off, _LANES)]
            t = (base + off) // _K + lane_tok
            idx_v[pl.ds(off, _LANES)] = sel * _ROW + t
            return carry

        lax.fori_loop(0, _CHUNK // _LANES, idx_body, 0)
        pltpu.sync_copy(rw_v, w_sh.at[idx_v], add=True)

    plsc.subcore_barrier()

    @pl.when((c == 0) & (s == 0))
    def _writeout():
        pltpu.sync_copy(w_sh, w_hbm)


def _routing_weights_sc(selected_experts, routing_weights):
    sel_flat = selected_experts.reshape(-1)
    rw_flat = routing_weights.reshape(-1)
    zeros = jnp.zeros((_E * _ROW,), jnp.float32)
    mesh = plsc.VectorSubcoreMesh(core_axis_name="c", subcore_axis_name="s",
                                  num_cores=1)
    w = pl.kernel(
        _routing_scatter_body,
        mesh=mesh,
        out_type=jax.ShapeDtypeStruct((_E * _ROW,), jnp.float32),
        scratch_types=[
            pltpu.VMEM((_CHUNK,), jnp.int32),
            pltpu.VMEM((_CHUNK,), jnp.float32),
            pltpu.VMEM((_CHUNK,), jnp.int32),
            pltpu.VMEM_SHARED((_E * _ROW,), jnp.float32),
        ],
    )(sel_flat, rw_flat, zeros)
    return w.reshape(_E, _ROW)


def _moe_body(hidden_ref, ident_ref, w_ref, gate_ref, up_ref, down_ref,
              out_ref):
    e = pl.program_id(0)
    x = hidden_ref[...]
    g = jax.lax.dot_general(x, gate_ref[...], (((1,), (1,)), ((), ())),
                            preferred_element_type=jnp.float32)
    u = jax.lax.dot_general(x, up_ref[...], (((1,), (1,)), ((), ())),
                            preferred_element_type=jnp.float32)
    h = g * jax.nn.sigmoid(g) * u
    d = jax.lax.dot_general(h, down_ref[...], (((1,), (1,)), ((), ())),
                            preferred_element_type=jnp.float32)
    wrow = w_ref[pl.ds(e, 1), 0:_T]
    wcol = jax.lax.dot_general(ident_ref[...], wrow,
                               (((1,), (1,)), ((), ())),
                               preferred_element_type=jnp.float32)
    contrib = wcol * d

    @pl.when(e == 0)
    def _init():
        out_ref[...] = contrib

    @pl.when(e != 0)
    def _acc():
        out_ref[...] += contrib


def kernel(hidden_states, routing_weights, selected_experts, num_experts,
           gate_proj, up_proj, down_proj):
    T, H = hidden_states.shape
    E, I, _ = gate_proj.shape
    w = _routing_weights_sc(selected_experts, routing_weights)
    ident = jnp.eye(T, dtype=jnp.float32)
    return pl.pallas_call(
        _moe_body,
        grid=(E,),
        in_specs=[
            pl.BlockSpec((T, H), lambda e: (0, 0)),
            pl.BlockSpec((T, T), lambda e: (0, 0)),
            pl.BlockSpec((E, _ROW), lambda e: (0, 0)),
            pl.BlockSpec((None, I, H), lambda e: (e, 0, 0)),
            pl.BlockSpec((None, I, H), lambda e: (e, 0, 0)),
            pl.BlockSpec((None, H, I), lambda e: (e, 0, 0)),
        ],
        out_specs=pl.BlockSpec((T, H), lambda e: (0, 0)),
        out_shape=jax.ShapeDtypeStruct((T, H), jnp.float32),
    )(hidden_states, ident, w, gate_proj, up_proj, down_proj)


# speedup vs baseline: 1.0520x; 1.0045x over previous
import jax
import jax.numpy as jnp
from jax import lax
from jax.experimental import pallas as pl
from jax.experimental.pallas import tpu as pltpu
from jax.experimental.pallas import tpu_sc as plsc

_T = 64
_K = 8
_E = 64
_LANES = 16
_ROW = 128

_CHUNK = 32
_NWORK = (_T * _K) // _CHUNK


_ZCHUNK = (_E * _ROW) // 16


def _routing_scatter_body(sel_hbm, rw_hbm, w_hbm,
                          sel_v, rw_v, idx_v, zero_v, w_sh):
    c = lax.axis_index("c")
    s = lax.axis_index("s")

    @pl.when(c == 0)
    def _zero():
        def zbody(i, carry):
            zero_v[pl.ds(i * _LANES, _LANES)] = jnp.zeros((_LANES,),
                                                          jnp.float32)
            return carry

        lax.fori_loop(0, _ZCHUNK // _LANES, zbody, 0)
        pltpu.sync_copy(zero_v, w_sh.at[pl.ds(s * _ZCHUNK, _ZCHUNK)])

    plsc.subcore_barrier()

    @pl.when((c == 0) & (s < _NWORK))
    def _scatter():
        base = s * _CHUNK
        pltpu.sync_copy(sel_hbm.at[pl.ds(base, _CHUNK)], sel_v)
        pltpu.sync_copy(rw_hbm.at[pl.ds(base, _CHUNK)], rw_v)

        lane = lax.broadcasted_iota(jnp.int32, (_LANES,), 0)
        lane_tok = jnp.where(lane >= _K, 1, 0)

        def idx_body(j, carry):
            off = j * _LANES
            sel = sel_v[pl.ds(off, _LANES)]
            t = (base + off) // _K + lane_tok
            idx_v[pl.ds(off, _LANES)] = sel * _ROW + t
            return carry

        lax.fori_loop(0, _CHUNK // _LANES, idx_body, 0)
        pltpu.sync_copy(rw_v, w_sh.at[idx_v], add=True)

    plsc.subcore_barrier()

    @pl.when((c == 0) & (s == 0))
    def _writeout():
        pltpu.sync_copy(w_sh, w_hbm)


def _routing_weights_sc(selected_experts, routing_weights):
    sel_flat = selected_experts.reshape(-1)
    rw_flat = routing_weights.reshape(-1)
    mesh = plsc.VectorSubcoreMesh(core_axis_name="c", subcore_axis_name="s",
                                  num_cores=1)
    w = pl.kernel(
        _routing_scatter_body,
        mesh=mesh,
        out_type=jax.ShapeDtypeStruct((_E * _ROW,), jnp.float32),
        scratch_types=[
            pltpu.VMEM((_CHUNK,), jnp.int32),
            pltpu.VMEM((_CHUNK,), jnp.float32),
            pltpu.VMEM((_CHUNK,), jnp.int32),
            pltpu.VMEM((_ZCHUNK,), jnp.float32),
            pltpu.VMEM_SHARED((_E * _ROW,), jnp.float32),
        ],
    )(sel_flat, rw_flat)
    return w.reshape(_E, _ROW)


def _moe_body(hidden_ref, ident_ref, w_ref, gate_ref, up_ref, down_ref,
              out_ref):
    e = pl.program_id(0)
    x = hidden_ref[...]
    g = jax.lax.dot_general(x, gate_ref[...], (((1,), (1,)), ((), ())),
                            preferred_element_type=jnp.float32)
    u = jax.lax.dot_general(x, up_ref[...], (((1,), (1,)), ((), ())),
                            preferred_element_type=jnp.float32)
    h = g * jax.nn.sigmoid(g) * u
    d = jax.lax.dot_general(h, down_ref[...], (((1,), (1,)), ((), ())),
                            preferred_element_type=jnp.float32)
    wrow = w_ref[pl.ds(e, 1), 0:_T]
    wcol = jax.lax.dot_general(ident_ref[...], wrow,
                               (((1,), (1,)), ((), ())),
                               preferred_element_type=jnp.float32)
    contrib = wcol * d

    @pl.when(e == 0)
    def _init():
        out_ref[...] = contrib

    @pl.when(e != 0)
    def _acc():
        out_ref[...] += contrib


def kernel(hidden_states, routing_weights, selected_experts, num_experts,
           gate_proj, up_proj, down_proj):
    T, H = hidden_states.shape
    E, I, _ = gate_proj.shape
    w = _routing_weights_sc(selected_experts, routing_weights)
    ident = jnp.eye(T, dtype=jnp.float32)
    return pl.pallas_call(
        _moe_body,
        grid=(E,),
        in_specs=[
            pl.BlockSpec((T, H), lambda e: (0, 0)),
            pl.BlockSpec((T, T), lambda e: (0, 0)),
            pl.BlockSpec((E, _ROW), lambda e: (0, 0)),
            pl.BlockSpec((None, I, H), lambda e: (e, 0, 0)),
            pl.BlockSpec((None, I, H), lambda e: (e, 0, 0)),
            pl.BlockSpec((None, H, I), lambda e: (e, 0, 0)),
        ],
        out_specs=pl.BlockSpec((T, H), lambda e: (0, 0)),
        out_shape=jax.ShapeDtypeStruct((T, H), jnp.float32),
    )(hidden_states, ident, w, gate_proj, up_proj, down_proj)
